# baseline (speedup 1.0000x reference)
import functools

import jax
import jax.numpy as jnp
from jax import lax
from jax.experimental import pallas as pl
from jax.experimental.pallas import tpu as pltpu
from jax.experimental.pallas import tpu_sc as plsc

BATCH = 16384
HIST = 200
K_SPLITS = 5
HK = HIST // K_SPLITS

NUM_CORES = 2
NUM_SUBCORES = 16
NUM_WORKERS = NUM_CORES * NUM_SUBCORES
PER_WORKER = BATCH // NUM_WORKERS
BCH = 128
NUM_CHUNKS = PER_WORKER // BCH
LANES = 16
SCALE = 2.0 / 3.0


def _sc_body(h0, x_hbm, y_hbm, out_hbm, xv, yv, outv):
    wid = lax.axis_index("s") * NUM_CORES + lax.axis_index("c")
    base = wid * PER_WORKER

    def chunk_body(c, carry):
        b0 = base + c * BCH
        pltpu.sync_copy(x_hbm.at[pl.ds(h0, HK), pl.ds(b0, BCH)], xv)
        pltpu.sync_copy(y_hbm.at[pl.ds(h0, HK), pl.ds(b0, BCH)], yv)

        def h_body(h, carry2):
            for bg in range(BCH // LANES):
                sl = pl.ds(bg * LANES, LANES)
                vx = xv[h, sl]
                vy = yv[h, sl]
                outv[0, h, sl] = vx.astype(jnp.float32) * SCALE - 1.0
                outv[1, h, sl] = vy.astype(jnp.float32) * SCALE - 1.0
            return carry2

        lax.fori_loop(0, HK, h_body, 0)
        pltpu.sync_copy(outv, out_hbm.at[:, :, pl.ds(b0, BCH)])
        return carry

    lax.fori_loop(0, NUM_CHUNKS, chunk_body, 0)


@jax.jit
def kernel(x_x, x_y):
    mesh = plsc.VectorSubcoreMesh(core_axis_name="c", subcore_axis_name="s")
    xt = x_x.T
    yt = x_y.T
    parts = []
    for k in range(K_SPLITS):
        val = pl.kernel(
            functools.partial(_sc_body, k * HK),
            out_type=jax.ShapeDtypeStruct((2, HK, BATCH), jnp.float32),
            mesh=mesh,
            scratch_types=[
                pltpu.VMEM((HK, BCH), jnp.int32),
                pltpu.VMEM((HK, BCH), jnp.int32),
                pltpu.VMEM((2, HK, BCH), jnp.float32),
            ],
            compiler_params=pltpu.CompilerParams(needs_layout_passes=False),
        )(xt, yt)
        parts.append(jnp.transpose(val, (2, 1, 0)).astype(jnp.complex64))
    return jnp.concatenate(parts, axis=1)


# speedup vs baseline: 1.0924x; 1.0924x over previous
import jax
import jax.numpy as jnp
from jax import lax
from jax.experimental import pallas as pl
from jax.experimental.pallas import tpu as pltpu
from jax.experimental.pallas import tpu_sc as plsc

BATCH = 16384
HIST = 200

NUM_CORES = 2
NUM_SUBCORES = 16
NUM_WORKERS = NUM_CORES * NUM_SUBCORES
PER_WORKER = BATCH // NUM_WORKERS
BCH = 128
NUM_CHUNKS = PER_WORKER // BCH
LANES = 16
SCALE = 2.0 / 3.0


def _sc_body(x_hbm, y_hbm, out_hbm, xv, yv, outv, semx, semy, semo):
    wid = lax.axis_index("s") * NUM_CORES + lax.axis_index("c")
    base = wid * PER_WORKER

    def in_copies(c):
        b0 = base + c * BCH
        cx = pltpu.make_async_copy(x_hbm.at[:, pl.ds(b0, BCH)], xv, semx)
        cy = pltpu.make_async_copy(y_hbm.at[:, pl.ds(b0, BCH)], yv, semy)
        return cx, cy

    def out_copy(c):
        b0 = base + c * BCH
        return pltpu.make_async_copy(outv, out_hbm.at[:, :, pl.ds(b0, BCH)], semo)

    cx, cy = in_copies(0)
    cx.start()
    cy.start()
    for c in range(NUM_CHUNKS):
        cx, cy = in_copies(c)
        cx.wait()
        cy.wait()
        if c > 0:
            out_copy(c - 1).wait()

        def h_body(h, carry2):
            for bg in range(BCH // LANES):
                sl = pl.ds(bg * LANES, LANES)
                vx = xv[h, sl]
                vy = yv[h, sl]
                outv[0, h, sl] = vx.astype(jnp.float32) * SCALE - 1.0
                outv[1, h, sl] = vy.astype(jnp.float32) * SCALE - 1.0
            return carry2

        lax.fori_loop(0, HIST, h_body, 0, unroll=2)
        out_copy(c).start()
        if c + 1 < NUM_CHUNKS:
            nx, ny = in_copies(c + 1)
            nx.start()
            ny.start()
    out_copy(NUM_CHUNKS - 1).wait()


@jax.jit
def kernel(x_x, x_y):
    mesh = plsc.VectorSubcoreMesh(core_axis_name="c", subcore_axis_name="s")
    val = pl.kernel(
        _sc_body,
        out_type=jax.ShapeDtypeStruct((2, HIST, BATCH), jnp.float32),
        mesh=mesh,
        scratch_types=[
            pltpu.VMEM((HIST, BCH), jnp.int32),
            pltpu.VMEM((HIST, BCH), jnp.int32),
            pltpu.VMEM((2, HIST, BCH), jnp.float32),
            pltpu.SemaphoreType.DMA,
            pltpu.SemaphoreType.DMA,
            pltpu.SemaphoreType.DMA,
        ],
        compiler_params=pltpu.CompilerParams(needs_layout_passes=False),
    )(x_x.T, x_y.T)
    return jnp.transpose(val, (2, 1, 0)).astype(jnp.complex64)
